# Initial kernel scaffold; baseline (speedup 1.0000x reference)
import jax
import jax.numpy as jnp
from jax.experimental import pallas as pl


def kernel(x, adj, W, b):
    raise NotImplementedError("write your pallas kernel here")


# speedup vs baseline: 1.0027x; 1.0027x over previous
import jax
import jax.numpy as jnp
from jax.experimental import pallas as pl
from jax.experimental.pallas import tpu as pltpu


def _pick_block_rows(n: int) -> int:
    best = 8
    for bm in range(8, min(n, 256) + 1, 8):
        if n % bm == 0:
            best = bm
    return best


def _gc_kernel(x_ref, w_ref, b_ref, adj_ref, out_ref, h_ref):
    @pl.when(pl.program_id(0) == 0)
    def _compute_h():
        h = jax.lax.dot_general(
            x_ref[...], w_ref[...],
            (((1,), (1,)), ((), ())),
            preferred_element_type=jnp.float32,
            precision=jax.lax.Precision.HIGHEST,
        ) + b_ref[...]
        h_ref[...] = h.astype(jnp.bfloat16)

    out_ref[...] = jnp.dot(
        adj_ref[...].astype(jnp.bfloat16), h_ref[...],
        preferred_element_type=jnp.float32,
    )


def kernel(x, adj, W, b):
    n, d_in = x.shape
    d_out = W.shape[0]
    bm = _pick_block_rows(n)
    grid = (n // bm,)
    return pl.pallas_call(
        _gc_kernel,
        grid=grid,
        in_specs=[
            pl.BlockSpec((n, d_in), lambda i: (0, 0)),
            pl.BlockSpec((d_out, d_in), lambda i: (0, 0)),
            pl.BlockSpec((1, d_out), lambda i: (0, 0)),
            pl.BlockSpec((bm, n), lambda i: (i, 0)),
        ],
        out_specs=pl.BlockSpec((bm, d_out), lambda i: (i, 0)),
        out_shape=jax.ShapeDtypeStruct((n, d_out), jnp.float32),
        scratch_shapes=[pltpu.VMEM((n, d_out), jnp.bfloat16)],
        compiler_params=pltpu.CompilerParams(
            dimension_semantics=("arbitrary",),
            vmem_limit_bytes=100 * 1024 * 1024,
        ),
    )(x, W, b.reshape(1, -1), adj)


# speedup vs baseline: 1.0350x; 1.0322x over previous
import jax
import jax.numpy as jnp
from jax.experimental import pallas as pl
from jax.experimental.pallas import tpu as pltpu


def _pick_block_rows(n: int) -> int:
    best = 8
    for bm in range(8, min(n, 448) + 1, 8):
        if n % bm == 0:
            best = bm
    return best


def _gc_kernel(x_ref, w_ref, b_ref, adj_ref, out_ref, h_ref):
    @pl.when(pl.program_id(0) == 0)
    def _compute_h():
        h = jax.lax.dot_general(
            x_ref[...], w_ref[...],
            (((1,), (1,)), ((), ())),
            preferred_element_type=jnp.float32,
        ) + b_ref[...]
        h_ref[...] = h.astype(jnp.bfloat16)

    out_ref[...] = jnp.dot(
        adj_ref[...].astype(jnp.bfloat16), h_ref[...],
        preferred_element_type=jnp.float32,
    )


def kernel(x, adj, W, b):
    n, d_in = x.shape
    d_out = W.shape[0]
    bm = _pick_block_rows(n)
    grid = (n // bm,)
    return pl.pallas_call(
        _gc_kernel,
        grid=grid,
        in_specs=[
            pl.BlockSpec((n, d_in), lambda i: (0, 0)),
            pl.BlockSpec((d_out, d_in), lambda i: (0, 0)),
            pl.BlockSpec((1, d_out), lambda i: (0, 0)),
            pl.BlockSpec((bm, n), lambda i: (i, 0)),
        ],
        out_specs=pl.BlockSpec((bm, d_out), lambda i: (i, 0)),
        out_shape=jax.ShapeDtypeStruct((n, d_out), jnp.float32),
        scratch_shapes=[pltpu.VMEM((n, d_out), jnp.bfloat16)],
        compiler_params=pltpu.CompilerParams(
            dimension_semantics=("arbitrary",),
            vmem_limit_bytes=100 * 1024 * 1024,
        ),
    )(x, W, b.reshape(1, -1), adj)
